# baseline (speedup 1.0000x reference)
import jax
import jax.numpy as jnp
from jax import lax
from jax.experimental import pallas as pl
from jax.experimental.pallas import tpu as pltpu
from jax.experimental.pallas import tpu_sc as plsc

_NC = 2
_NS = 16
_NW = _NC * _NS
_CH = 128
_GPS = 2
_SCH = _CH * _GPS
_NBUF = 3
_PF = 2


def _gather_sc(table, idx):
    V, D = table.shape
    assert V % 8 == 0
    B = idx.shape[0]
    assert B % (_NW * _SCH) == 0
    per_w = B // _NW
    n_ch = per_w // _CH
    n_sc = per_w // _SCH
    q = _NBUF - _PF
    assert q >= 1 and n_sc >= _NBUF + _PF

    mesh = plsc.VectorSubcoreMesh(core_axis_name="c", subcore_axis_name="s")

    def body(table_hbm, idx_hbm, out_hbm, idx_v, table_sp, *bufs):
        rows = bufs[:_NBUF]
        gsem = bufs[_NBUF:2 * _NBUF]
        osem = bufs[2 * _NBUF:3 * _NBUF]
        sid = lax.axis_index("s")
        wid = sid * _NC + lax.axis_index("c")
        base = wid * per_w

        rpt = (-(-V // _NS) + 7) // 8 * 8
        n_full = V // rpt
        rem = V - n_full * rpt

        @pl.when(sid < n_full)
        def _():
            pltpu.sync_copy(
                table_hbm.at[pl.ds(sid * rpt, rpt)],
                table_sp.at[pl.ds(sid * rpt, rpt)])

        if rem:
            @pl.when(sid == n_full)
            def _():
                pltpu.sync_copy(
                    table_hbm.at[pl.ds(n_full * rpt, rem)],
                    table_sp.at[pl.ds(n_full * rpt, rem)])

        pltpu.sync_copy(idx_hbm.at[wid], idx_v)
        plsc.subcore_barrier()

        def gathers(j, b):
            return [
                pltpu.make_async_copy(
                    table_sp.at[idx_v.at[j * _GPS + g]],
                    rows[b].at[pl.ds(g * _CH, _CH)], gsem[b])
                for g in range(_GPS)
            ]

        def store(j, b):
            return pltpu.make_async_copy(
                rows[b], out_hbm.at[pl.ds(base + j * _SCH, _SCH)], osem[b])

        def start_gathers(j, b):
            for c in gathers(j, b):
                c.start()

        def wait_gathers(j, b):
            for c in gathers(j, b):
                c.wait()

        for j in range(_PF):
            start_gathers(j, j % _NBUF)
        for j in range(q):
            wait_gathers(j, j % _NBUF)
            store(j, j % _NBUF).start()
            start_gathers(j + _PF, (j + _PF) % _NBUF)

        def iter_body(j, b, bg):
            wait_gathers(j, b)
            store(j, b).start()
            store(j - q, bg).wait()
            start_gathers(j + _PF, bg)

        n_main = (n_sc - _PF - q) // _NBUF

        def step(i, _):
            j0 = q + i * _NBUF
            for u in range(_NBUF):
                iter_body(j0 + u, (q + u) % _NBUF, u % _NBUF)
            return _

        lax.fori_loop(0, n_main, step, 0)
        for j in range(q + n_main * _NBUF, n_sc - _PF):
            iter_body(j, j % _NBUF, (j - q) % _NBUF)

        for j in range(n_sc - _PF, n_sc):
            wait_gathers(j, j % _NBUF)
            store(j, j % _NBUF).start()
        for j in range(n_sc - _NBUF, n_sc):
            store(j, j % _NBUF).wait()

    f = pl.kernel(
        body,
        out_type=jax.ShapeDtypeStruct((B, D), jnp.float32),
        mesh=mesh,
        scratch_types=(
            [pltpu.VMEM((n_ch, _CH), jnp.int32),
             pltpu.VMEM_SHARED((V, D), jnp.float32)]
            + [pltpu.VMEM((_SCH, D), jnp.float32) for _ in range(_NBUF)]
            + [pltpu.SemaphoreType.DMA for _ in range(2 * _NBUF)]
        ),
    )
    return f(table, idx.reshape(_NW, n_ch, _CH))


def kernel(gene_pos, pe):
    B, L = gene_pos.shape
    table = pe.reshape(pe.shape[0], pe.shape[-1])
    flat = gene_pos.reshape(-1)
    out = _gather_sc(table, flat)
    return out.reshape(B, L, pe.shape[-1])


# speedup vs baseline: 1.0884x; 1.0884x over previous
import jax
import jax.numpy as jnp
from jax import lax
from jax.experimental import pallas as pl
from jax.experimental.pallas import tpu as pltpu
from jax.experimental.pallas import tpu_sc as plsc

_NC = 2
_NS = 16
_NW = _NC * _NS
_CH = 128
_NBUF = 5
_PF = 3


def _gather_sc(table, idx):
    V, D = table.shape
    assert V % 8 == 0
    B = idx.shape[0]
    assert B % (_NW * _CH) == 0
    per_w = B // _NW
    n_ch = per_w // _CH
    assert n_ch % _NBUF == 0 and n_ch >= 2 * _NBUF
    q = _NBUF - _PF

    mesh = plsc.VectorSubcoreMesh(core_axis_name="c", subcore_axis_name="s")

    def body(table_hbm, idx_hbm, out_hbm, idx_v, table_sp, *bufs):
        rows = bufs[:_NBUF]
        gsem = bufs[_NBUF:2 * _NBUF]
        osem = bufs[2 * _NBUF:3 * _NBUF]
        sid = lax.axis_index("s")
        wid = lax.axis_index("c") * _NS + sid
        base = wid * per_w

        rpt = (-(-V // _NS) + 7) // 8 * 8
        n_full = V // rpt
        rem = V - n_full * rpt

        @pl.when(sid < n_full)
        def _():
            pltpu.sync_copy(
                table_hbm.at[pl.ds(sid * rpt, rpt)],
                table_sp.at[pl.ds(sid * rpt, rpt)])

        if rem:
            @pl.when(sid == n_full)
            def _():
                pltpu.sync_copy(
                    table_hbm.at[pl.ds(n_full * rpt, rem)],
                    table_sp.at[pl.ds(n_full * rpt, rem)])

        pltpu.sync_copy(idx_hbm.at[wid], idx_v)
        plsc.subcore_barrier()

        def gather(j, b):
            return pltpu.make_async_copy(
                table_sp.at[idx_v.at[j]], rows[b], gsem[b])

        def store(j, b):
            return pltpu.make_async_copy(
                rows[b], out_hbm.at[pl.ds(base + j * _CH, _CH)], osem[b])

        for j in range(_PF):
            gather(j, j % _NBUF).start()
        for j in range(q):
            gather(j, j % _NBUF).wait()
            store(j, j % _NBUF).start()
            gather(j + _PF, (j + _PF) % _NBUF).start()

        def step(i, _):
            j0 = q + i * _NBUF
            for u in range(_NBUF):
                j = j0 + u
                b = (q + u) % _NBUF
                gather(j, b).wait()
                store(j, b).start()
                bg = u % _NBUF
                store(j - q, bg).wait()
                gather(j + _PF, bg).start()
            return _

        n_main = (n_ch - _PF - q) // _NBUF
        lax.fori_loop(0, n_main, step, 0)

        for j in range(n_ch - _PF, n_ch):
            gather(j, j % _NBUF).wait()
            store(j, j % _NBUF).start()
        for j in range(n_ch - _NBUF, n_ch):
            store(j, j % _NBUF).wait()

    f = pl.kernel(
        body,
        out_type=jax.ShapeDtypeStruct((B, D), jnp.float32),
        mesh=mesh,
        scratch_types=(
            [pltpu.VMEM((n_ch, _CH), jnp.int32),
             pltpu.VMEM_SHARED((V, D), jnp.float32)]
            + [pltpu.VMEM((_CH, D), jnp.float32) for _ in range(_NBUF)]
            + [pltpu.SemaphoreType.DMA for _ in range(2 * _NBUF)]
        ),
    )
    return f(table, idx.reshape(_NW, n_ch, _CH))


def kernel(gene_pos, pe):
    B, L = gene_pos.shape
    table = pe.reshape(pe.shape[0], pe.shape[-1])
    flat = gene_pos.reshape(-1)
    out = _gather_sc(table, flat)
    return out.reshape(B, L, pe.shape[-1])


# speedup vs baseline: 1.0923x; 1.0036x over previous
import jax
import jax.numpy as jnp
from jax import lax
from jax.experimental import pallas as pl
from jax.experimental.pallas import tpu as pltpu
from jax.experimental.pallas import tpu_sc as plsc

_NC = 2
_NS = 16
_NW = _NC * _NS
_CH = 128
_NBUF = 5
_PF = 3


def _gather_sc(table, idx):
    V, D = table.shape
    assert V % 8 == 0
    B = idx.shape[0]
    assert B % (_NW * _CH) == 0
    per_w = B // _NW
    n_ch = per_w // _CH
    assert n_ch % _NBUF == 0 and n_ch >= 2 * _NBUF
    q = _NBUF - _PF

    mesh = plsc.VectorSubcoreMesh(core_axis_name="c", subcore_axis_name="s")

    def body(table_hbm, idx_hbm, out_hbm, idx_v, table_sp, *bufs):
        rows = bufs[:_NBUF]
        gsem = bufs[_NBUF:2 * _NBUF]
        osem = bufs[2 * _NBUF:3 * _NBUF]
        sid = lax.axis_index("s")
        wid = lax.axis_index("c") * _NS + sid
        base = wid * per_w

        rpt = (-(-V // _NS) + 7) // 8 * 8
        n_full = V // rpt
        rem = V - n_full * rpt

        @pl.when(sid < n_full)
        def _():
            pltpu.sync_copy(
                table_hbm.at[pl.ds(sid * rpt, rpt)],
                table_sp.at[pl.ds(sid * rpt, rpt)])

        if rem:
            @pl.when(sid == n_full)
            def _():
                pltpu.sync_copy(
                    table_hbm.at[pl.ds(n_full * rpt, rem)],
                    table_sp.at[pl.ds(n_full * rpt, rem)])

        pltpu.sync_copy(idx_hbm.at[wid], idx_v)
        plsc.subcore_barrier()

        def gather(j, b):
            return pltpu.make_async_copy(
                table_sp.at[idx_v.at[j]], rows[b], gsem[b])

        def store(j, b):
            return pltpu.make_async_copy(
                rows[b], out_hbm.at[pl.ds(base + j * _CH, _CH)], osem[b])

        for j in range(_PF):
            gather(j, j % _NBUF).start()
        for j in range(q):
            gather(j, j % _NBUF).wait()
            store(j, j % _NBUF).start()
            gather(j + _PF, (j + _PF) % _NBUF).start()

        def step(i, _):
            j0 = q + i * _NBUF
            for u in range(_NBUF):
                j = j0 + u
                b = (q + u) % _NBUF
                bg = u % _NBUF
                store(j - q, bg).wait()
                gather(j + _PF, bg).start()
                gather(j, b).wait()
                store(j, b).start()
            return _

        n_main = (n_ch - _PF - q) // _NBUF
        lax.fori_loop(0, n_main, step, 0)

        for j in range(n_ch - _PF, n_ch):
            gather(j, j % _NBUF).wait()
            store(j, j % _NBUF).start()
        for j in range(n_ch - _NBUF, n_ch):
            store(j, j % _NBUF).wait()

    f = pl.kernel(
        body,
        out_type=jax.ShapeDtypeStruct((B, D), jnp.float32),
        mesh=mesh,
        scratch_types=(
            [pltpu.VMEM((n_ch, _CH), jnp.int32),
             pltpu.VMEM_SHARED((V, D), jnp.float32)]
            + [pltpu.VMEM((_CH, D), jnp.float32) for _ in range(_NBUF)]
            + [pltpu.SemaphoreType.DMA for _ in range(2 * _NBUF)]
        ),
    )
    return f(table, idx.reshape(_NW, n_ch, _CH))


def kernel(gene_pos, pe):
    B, L = gene_pos.shape
    table = pe.reshape(pe.shape[0], pe.shape[-1])
    flat = gene_pos.reshape(-1)
    out = _gather_sc(table, flat)
    return out.reshape(B, L, pe.shape[-1])
